# baseline (speedup 1.0000x reference)
import functools

import jax
import jax.numpy as jnp
from jax import lax
from jax.experimental import pallas as pl
from jax.experimental.pallas import tpu as pltpu
from jax.experimental.pallas import tpu_sc as plsc

NUM_CORES = 2
NUM_SUBCORES = 16
NUM_WORKERS = NUM_CORES * NUM_SUBCORES

NUM_CLASSES = 1000000
BATCH = 16384
HIDDEN = 64

TCOLS = 7813
TAIL_TCOL = 7812
TAIL_BASE = TAIL_TCOL * 128
BASE_TCOLS = TCOLS // NUM_WORKERS
EXTRA = TCOLS - BASE_TCOLS * NUM_WORKERS
CAP = BATCH + 16
SB = 4
OUT_WORDS = BATCH * HIDDEN
PAD_BASE = OUT_WORDS

_mesh = plsc.VectorSubcoreMesh(core_axis_name="c", subcore_axis_name="s")


@functools.partial(
    pl.kernel,
    mesh=_mesh,
    out_type=jax.ShapeDtypeStruct((OUT_WORDS + 128,), jnp.float32),
    scratch_types=[
        pltpu.VMEM((BATCH,), jnp.int32),
        pltpu.VMEM((CAP,), jnp.int32),
        pltpu.VMEM((CAP,), jnp.int32),
        pltpu.VMEM((SB * HIDDEN, 128), jnp.float32),
        pltpu.VMEM((SB * HIDDEN, 128), jnp.float32),
        pltpu.VMEM((32,), jnp.int32),
        pltpu.VMEM((32,), jnp.int32),
        pltpu.VMEM((2 * 16 * HIDDEN,), jnp.float32),
        pltpu.VMEM((16, 128), jnp.int32),
        pltpu.SemaphoreType.DMA,
        pltpu.SemaphoreType.DMA,
        pltpu.SemaphoreType.DMA,
        pltpu.SemaphoreType.DMA,
    ],
    compiler_params=pltpu.CompilerParams(
        use_tc_tiling_on_sc=True, needs_layout_passes=False
    ),
)
def _sc_gather(
    idx_hbm,
    table_hbm,
    tail_hbm,
    out_hbm,
    idx_v,
    hitlab_v,
    hitpos_v,
    stage_a,
    stage_b,
    grel_v,
    gpos_v,
    gbt_v,
    idxb_v,
    sem_a,
    sem_b,
    osem0,
    osem1,
):
    wid = lax.axis_index("s") * NUM_CORES + lax.axis_index("c")
    start_tc = wid * BASE_TCOLS + jnp.minimum(wid, EXTRA)
    n_tc = jnp.where(wid < EXTRA, BASE_TCOLS + 1, BASE_TCOLS)
    end_tc = start_tc + n_tc
    lo = start_tc * 128
    hi = end_tc * 128
    nsb = (n_tc + SB - 1) // SB

    def issue_stage(stage, sem, sb0_tc):
        for t in range(SB):
            gid = sb0_tc + t
            safe = jnp.minimum(gid, TAIL_TCOL - 1)
            cbase = pl.multiple_of(safe * 128, 128)
            band = stage.at[pl.ds(t * HIDDEN, HIDDEN), :]

            @pl.when(gid != TAIL_TCOL)
            def _():
                pltpu.async_copy(
                    table_hbm.at[:, pl.ds(cbase, 128)], band, sem
                )

            @pl.when(gid == TAIL_TCOL)
            def _():
                pltpu.async_copy(tail_hbm, band, sem)

    def wait_stage(stage, sem):
        for _ in range(SB):
            pltpu.make_async_copy(
                table_hbm.at[:, pl.ds(0, 128)],
                stage.at[pl.ds(0, HIDDEN), :],
                sem,
            ).wait()

    issue_stage(stage_a, sem_a, start_tc)

    @pl.when(nsb > 1)
    def _():
        issue_stage(stage_b, sem_b, start_tc + SB)

    pltpu.sync_copy(idx_hbm, idx_v)

    def select(v4, nhits):
        for u in range(4):
            v = v4 * 4 + u
            labv = idx_v[pl.ds(v * 16, 16)]
            posv = lax.iota(jnp.int32, 16) + (v * 16)
            m = (labv >= lo) & (labv < hi)
            plsc.store_compressed(hitlab_v.at[pl.ds(nhits, 16)], labv, mask=m)
            plsc.store_compressed(hitpos_v.at[pl.ds(nhits, 16)], posv, mask=m)
            nhits = nhits + plsc.all_reduce_population_count(m)[0]
        return nhits

    nhits = lax.fori_loop(0, BATCH // 64, select, jnp.int32(0))
    nchunks = (nhits + 15) >> 4

    def drain_scatter(half):
        pltpu.make_async_copy(
            out_hbm.at[pl.ds(0, 16 * HIDDEN)],
            gbt_v.at[pl.ds(half * 16 * HIDDEN, 16 * HIDDEN)],
            [osem0, osem1][half],
        ).wait()

    def flush_into(stage, half, osem, gcount):
        relv = grel_v[pl.ds(0, 16)]
        posg = gpos_v[pl.ds(0, 16)]
        lanes = lax.iota(jnp.int32, 16)
        lm = lanes < gcount
        rbase = (relv >> 7) * HIDDEN
        cc = relv & 127
        pos64 = posg * HIDDEN
        padv = PAD_BASE + lanes
        goff = pl.multiple_of(half * 16 * HIDDEN, 16)
        irow0 = half * 8

        def launch(q, _):
            joff = q * 8
            for r in range(8):
                j = joff + r
                vals = plsc.load_gather(
                    stage, [rbase + j, cc], mask=lm
                )
                gbt_v[pl.ds(goff + (q * 128 + r * 16), 16)] = vals
                idxr = jnp.where(lm, pos64 + j, padv)
                idxb_v[irow0 + q, pl.ds(r * 16, 16)] = idxr
            pltpu.async_copy(
                gbt_v.at[pl.ds(goff + q * 128, 128)],
                out_hbm.at[idxb_v.at[irow0 + q]],
                osem,
            )
            return ()

        lax.fori_loop(0, 8, launch, ())

    def do_flush(stage, nf, gcount):
        @pl.when(nf >= 2)
        def _():
            lax.cond(
                (nf & 1) == 0,
                lambda: drain_scatter(0),
                lambda: drain_scatter(1),
            )

        lax.cond(
            (nf & 1) == 0,
            lambda: flush_into(stage, 0, osem0, gcount),
            lambda: flush_into(stage, 1, osem1, gcount),
        )

    def emit(stage, sb, carry):
        gc, nf = carry
        blo = (start_tc + sb * SB) * 128
        bhi = jnp.minimum(blo + SB * 128, hi)

        def chunk(t, carry):
            gc, nf = carry
            labv = hitlab_v[pl.ds(t * 16, 16)]
            valid = (lax.iota(jnp.int32, 16) + t * 16) < nhits
            m = valid & (labv >= blo) & (labv < bhi)
            npc = plsc.all_reduce_population_count(m)[0]

            def add_hits(carry):
                gc, nf = carry
                posv = hitpos_v[pl.ds(t * 16, 16)]
                rel = labv - blo
                plsc.store_compressed(grel_v.at[pl.ds(gc, 16)], rel, mask=m)
                plsc.store_compressed(gpos_v.at[pl.ds(gc, 16)], posv, mask=m)
                gc = gc + npc

                def full(carry):
                    gc, nf = carry
                    do_flush(stage, nf, jnp.int32(16))
                    r2 = grel_v[pl.ds(16, 16)]
                    p2 = gpos_v[pl.ds(16, 16)]
                    grel_v[pl.ds(0, 16)] = r2
                    gpos_v[pl.ds(0, 16)] = p2
                    return (gc - 16, nf + 1)

                return lax.cond(gc >= 16, full, lambda c: c, (gc, nf))

            return lax.cond(npc > 0, add_hits, lambda c: c, (gc, nf))

        gc, nf = lax.fori_loop(0, nchunks, chunk, (gc, nf))

        def partial(carry):
            gc, nf = carry
            do_flush(stage, nf, gc)
            return (jnp.int32(0), nf + 1)

        return lax.cond(gc > 0, partial, lambda c: c, (gc, nf))

    def pair(k2, carry):
        sba = 2 * k2
        wait_stage(stage_a, sem_a)
        carry = emit(stage_a, sba, carry)

        @pl.when(sba + 2 < nsb)
        def _():
            issue_stage(stage_a, sem_a, start_tc + (sba + 2) * SB)

        @pl.when(sba + 1 < nsb)
        def _():
            wait_stage(stage_b, sem_b)

        carry = lax.cond(
            sba + 1 < nsb,
            lambda c: emit(stage_b, sba + 1, c),
            lambda c: c,
            carry,
        )

        @pl.when(sba + 3 < nsb)
        def _():
            issue_stage(stage_b, sem_b, start_tc + (sba + 3) * SB)

        return carry

    _, nf = lax.fori_loop(
        0, (nsb + 1) >> 1, pair, (jnp.int32(0), jnp.int32(0))
    )

    @pl.when(nf >= 1)
    def _():
        lax.cond(
            ((nf - 1) & 1) == 0,
            lambda: drain_scatter(0),
            lambda: drain_scatter(1),
        )

    @pl.when(nf >= 2)
    def _():
        lax.cond(
            ((nf - 2) & 1) == 0,
            lambda: drain_scatter(0),
            lambda: drain_scatter(1),
        )


def kernel(labels, embedding):
    idx = labels.astype(jnp.int32)
    tail = jnp.pad(embedding[TAIL_BASE:].T, ((0, 0), (0, 128 - 64)))
    out1d = _sc_gather(idx, embedding.T, tail)
    return out1d[:OUT_WORDS].reshape(BATCH, HIDDEN)


# speedup vs baseline: 539.0798x; 539.0798x over previous
import functools

import jax
import jax.numpy as jnp
from jax import lax
from jax.experimental import pallas as pl
from jax.experimental.pallas import tpu as pltpu
from jax.experimental.pallas import tpu_sc as plsc

NUM_CORES = 2
NUM_SUBCORES = 16
NUM_WORKERS = NUM_CORES * NUM_SUBCORES

NUM_CLASSES = 1000000
BATCH = 16384
HIDDEN = 64

TCOLS = 7813
TAIL_TCOL = 7812
TAIL_BASE = TAIL_TCOL * 128
BASE_TCOLS = TCOLS // NUM_WORKERS
EXTRA = TCOLS - BASE_TCOLS * NUM_WORKERS
CAP = BATCH + 16
SB = 4

_mesh = plsc.VectorSubcoreMesh(core_axis_name="c", subcore_axis_name="s")


@functools.partial(
    pl.kernel,
    mesh=_mesh,
    out_type=(
        jax.ShapeDtypeStruct((BATCH * HIDDEN,), jnp.float32),
        jax.ShapeDtypeStruct((16 * HIDDEN,), jnp.float32),
    ),
    scratch_types=[
        pltpu.VMEM((BATCH,), jnp.int32),
        pltpu.VMEM((CAP,), jnp.int32),
        pltpu.VMEM((CAP,), jnp.int32),
        pltpu.VMEM((SB * HIDDEN, 128), jnp.float32),
        pltpu.VMEM((SB * HIDDEN, 128), jnp.float32),
        pltpu.VMEM((32 * HIDDEN,), jnp.float32),
        pltpu.SemaphoreType.DMA,
        pltpu.SemaphoreType.DMA,
        pltpu.SemaphoreType.DMA,
        pltpu.SemaphoreType.DMA,
    ],
    compiler_params=pltpu.CompilerParams(
        use_tc_tiling_on_sc=True, needs_layout_passes=False
    ),
)
def _sc_gather(
    idx_hbm,
    table_hbm,
    tail_hbm,
    out_hbm,
    dump_hbm,
    idx_v,
    hitlab_v,
    hitpos_v,
    stage_a,
    stage_b,
    ring_v,
    sem_a,
    sem_b,
    osem0,
    osem1,
):
    wid = lax.axis_index("s") * NUM_CORES + lax.axis_index("c")
    start_tc = wid * BASE_TCOLS + jnp.minimum(wid, EXTRA)
    n_tc = jnp.where(wid < EXTRA, BASE_TCOLS + 1, BASE_TCOLS)
    lo = start_tc * 128
    hi = (start_tc + n_tc) * 128
    nsb = (n_tc + SB - 1) // SB

    def issue_stage(stage, sem, sb0_tc):
        for t in range(SB):
            gid = sb0_tc + t
            safe = jnp.minimum(gid, TAIL_TCOL - 1)
            cbase = pl.multiple_of(safe * 128, 128)
            band = stage.at[pl.ds(t * HIDDEN, HIDDEN), :]

            @pl.when(gid != TAIL_TCOL)
            def _():
                pltpu.async_copy(
                    table_hbm.at[:, pl.ds(cbase, 128)], band, sem
                )

            @pl.when(gid == TAIL_TCOL)
            def _():
                pltpu.async_copy(tail_hbm, band, sem)

    def wait_stage(stage, sem):
        for _ in range(SB):
            pltpu.make_async_copy(
                table_hbm.at[:, pl.ds(0, 128)],
                stage.at[pl.ds(0, HIDDEN), :],
                sem,
            ).wait()

    issue_stage(stage_a, sem_a, start_tc)

    @pl.when(nsb > 1)
    def _():
        issue_stage(stage_b, sem_b, start_tc + SB)

    pltpu.sync_copy(idx_hbm, idx_v)

    def select(v4, nhits):
        for u in range(4):
            v = v4 * 4 + u
            labv = idx_v[pl.ds(v * 16, 16)]
            posv = lax.iota(jnp.int32, 16) + (v * 16)
            m = (labv >= lo) & (labv < hi)
            plsc.store_compressed(hitlab_v.at[pl.ds(nhits, 16)], labv, mask=m)
            plsc.store_compressed(hitpos_v.at[pl.ds(nhits, 16)], posv, mask=m)
            nhits = nhits + plsc.all_reduce_population_count(m)[0]
        return nhits

    nhits = lax.fori_loop(0, BATCH // 64, select, jnp.int32(0))
    nchunks = (nhits + 15) >> 4

    def drain_pool(p):
        def mk(sem):
            pltpu.make_async_copy(
                out_hbm.at[pl.ds(0, 16 * HIDDEN)],
                ring_v.at[pl.ds(p * 16 * HIDDEN, 16 * HIDDEN)],
                sem,
            ).wait()

        lax.cond(p == 0, lambda: mk(osem0), lambda: mk(osem1))

    def out_copy(slot, pos):
        soff = pl.multiple_of(slot * HIDDEN, HIDDEN)
        off = pl.multiple_of(pos * HIDDEN, HIDDEN)

        def mk(sem):
            pltpu.async_copy(
                ring_v.at[pl.ds(soff, HIDDEN)],
                out_hbm.at[pl.ds(off, HIDDEN)],
                sem,
            )

        lax.cond((slot >> 4) == 0, lambda: mk(osem0), lambda: mk(osem1))

    def emit(stage, sb, hcnt):
        blo = (start_tc + sb * SB) * 128
        bhi = jnp.minimum(blo + SB * 128, hi)

        def chunk(t, hcnt):
            labv = hitlab_v[pl.ds(t * 16, 16)]
            valid = (lax.iota(jnp.int32, 16) + t * 16) < nhits
            m = valid & (labv >= blo) & (labv < bhi)
            npc = plsc.all_reduce_population_count(m)[0]

            def lanes(hcnt):
                posv = hitpos_v[pl.ds(t * 16, 16)]
                mi = m.astype(jnp.int32)
                for h in range(16):
                    slot = hcnt & 31
                    hit = mi[h]

                    @pl.when(hit == 1)
                    def _():
                        @pl.when(((slot & 15) == 0) & (hcnt >= 32))
                        def _():
                            drain_pool(slot >> 4)

                        rel = labv[h] - blo
                        c = rel & 127
                        rbase = (rel >> 7) * HIDDEN
                        cols = jnp.full((16,), c, jnp.int32)
                        soff = pl.multiple_of(slot * HIDDEN, HIDDEN)
                        for j4 in range(4):
                            rows = lax.iota(jnp.int32, 16) + (16 * j4 + rbase)
                            vals = plsc.load_gather(stage, [rows, cols])
                            ring_v[pl.ds(soff + 16 * j4, 16)] = vals
                        out_copy(slot, posv[h])

                    hcnt = hcnt + hit
                return hcnt

            return lax.cond(npc > 0, lanes, lambda hcnt: hcnt, hcnt)

        return lax.fori_loop(0, nchunks, chunk, hcnt)

    def pair(k2, hcnt):
        sba = 2 * k2
        wait_stage(stage_a, sem_a)
        hcnt = emit(stage_a, sba, hcnt)

        @pl.when(sba + 2 < nsb)
        def _():
            issue_stage(stage_a, sem_a, start_tc + (sba + 2) * SB)

        @pl.when(sba + 1 < nsb)
        def _():
            wait_stage(stage_b, sem_b)

        hcnt = lax.cond(
            sba + 1 < nsb,
            lambda hcnt: emit(stage_b, sba + 1, hcnt),
            lambda hcnt: hcnt,
            hcnt,
        )

        @pl.when(sba + 3 < nsb)
        def _():
            issue_stage(stage_b, sem_b, start_tc + (sba + 3) * SB)

        return hcnt

    hcnt = lax.fori_loop(0, (nsb + 1) >> 1, pair, jnp.int32(0))

    npad = (16 - (hcnt & 15)) & 15

    def pad(_, hcnt):
        slot = hcnt & 31

        @pl.when(((slot & 15) == 0) & (hcnt >= 32))
        def _():
            drain_pool(slot >> 4)

        soff = pl.multiple_of(slot * HIDDEN, HIDDEN)

        def mk(sem):
            pltpu.async_copy(
                ring_v.at[pl.ds(soff, HIDDEN)],
                dump_hbm.at[
                    pl.ds(
                        pl.multiple_of(soff & (16 * HIDDEN - 1), HIDDEN),
                        HIDDEN,
                    )
                ],
                sem,
            )

        lax.cond((slot >> 4) == 0, lambda: mk(osem0), lambda: mk(osem1))
        return hcnt + 1

    hcnt = lax.fori_loop(0, npad, pad, hcnt)

    @pl.when(hcnt >= 16)
    def _():
        drain_pool(((hcnt - 16) >> 4) & 1)

    @pl.when(hcnt >= 32)
    def _():
        drain_pool(((hcnt - 32) >> 4) & 1)


def kernel(labels, embedding):
    idx = labels.astype(jnp.int32)
    tail = jnp.pad(embedding[TAIL_BASE:].T, ((0, 0), (0, 128 - 64)))
    out1d, _ = _sc_gather(idx, embedding.T, tail)
    return out1d.reshape(BATCH, HIDDEN)
